# baseline (speedup 1.0000x reference)
import functools

import jax
import jax.numpy as jnp
from jax import lax
from jax.experimental import pallas as pl
from jax.experimental.pallas import tpu as pltpu
from jax.experimental.pallas import tpu_sc as plsc

N = 10000
D = 128
E = 320000
Z = 64
H = 64
C = 10
CP = 16

NC = 2
NS = 16
NW = NC * NS
EPW = E // NW
K = 100
NCH = EPW // K
WBT = 10
RPT = N // WBT
RCH = 200
NRC = RPT // RCH
GRP = 10
NGRP = NCH // GRP

_mesh = plsc.VectorSubcoreMesh(core_axis_name="c", subcore_axis_name="s")


@functools.partial(
    pl.kernel,
    out_type=jax.ShapeDtypeStruct((2 * N,), jnp.float32),
    mesh=_mesh,
    scratch_types=[
        pltpu.VMEM((NGRP, GRP, K), jnp.int32),
        pltpu.VMEM((128,), jnp.float32),
        pltpu.VMEM((1024,), jnp.float32),
        pltpu.VMEM_SHARED((N,), jnp.float32),
        pltpu.SemaphoreType.DMA,
    ],
)
def _sc_deg(eidx_hbm, out_hbm, didx, ones, zbuf, acc, dsem):
    cid = lax.axis_index("c")
    sid = lax.axis_index("s")
    wid = cid * NS + sid

    one16 = jnp.ones((16,), jnp.float32)
    zero16 = jnp.zeros((16,), jnp.float32)
    for j in range(8):
        ones[pl.ds(j * 16, 16)] = one16

    @pl.loop(0, 64)
    def _zero(i):
        zbuf[pl.ds(i * 16, 16)] = zero16

    @pl.when(sid < 10)
    def _():
        pltpu.sync_copy(zbuf.at[pl.ds(0, 1000)], acc.at[pl.ds(sid * 1000, 1000)])

    plsc.subcore_barrier()

    pltpu.sync_copy(eidx_hbm.at[1, wid], didx)

    @pl.loop(0, NGRP)
    def _edges(g):
        for j in range(GRP):
            pltpu.async_copy(ones.at[pl.ds(0, K)], acc.at[didx.at[g, j]],
                             dsem, add=True)
        for j in range(GRP):
            pltpu.make_async_copy(ones.at[pl.ds(0, K)], acc.at[didx.at[g, j]],
                                  dsem).wait()

    plsc.subcore_barrier()

    @pl.when(sid < 10)
    def _():
        pltpu.sync_copy(acc.at[pl.ds(sid * 1000, 1000)], zbuf.at[pl.ds(0, 1000)])
        pltpu.sync_copy(zbuf.at[pl.ds(0, 1000)],
                        out_hbm.at[pl.ds(cid * N + sid * 1000, 1000)])


@functools.partial(
    pl.kernel,
    out_type=jax.ShapeDtypeStruct((2 * N, D), jnp.float32),
    mesh=_mesh,
    scratch_types=[
        pltpu.VMEM((2, GRP, K), jnp.int32),
        pltpu.VMEM((2, GRP, K), jnp.int32),
        pltpu.VMEM((K, D), jnp.float32),
        pltpu.VMEM((K, D), jnp.float32),
        pltpu.VMEM((K, D), jnp.float32),
        pltpu.VMEM_SHARED((N, D), jnp.float32),
        pltpu.SemaphoreType.DMA,
        pltpu.SemaphoreType.DMA,
        pltpu.SemaphoreType.DMA,
        pltpu.SemaphoreType.DMA,
        pltpu.SemaphoreType.DMA,
    ],
)
def _sc_edge(h_hbm, eidx_hbm, z_hbm, out_hbm, sring, dring, rows0, rows1,
             rows2, acc, sem0, sem1, sem2, rsem0, rsem1):
    cid = lax.axis_index("c")
    sid = lax.axis_index("s")
    wid = cid * NS + sid

    ir0 = sid * 664

    @pl.when(cid == 0)
    def _():
        @pl.when(sid < NS - 1)
        def _():
            pltpu.sync_copy(h_hbm.at[pl.ds(ir0, 664)], acc.at[pl.ds(ir0, 664)])

        @pl.when(sid == NS - 1)
        def _():
            pltpu.sync_copy(h_hbm.at[pl.ds(ir0, 40)], acc.at[pl.ds(ir0, 40)])

    @pl.when(cid == 1)
    def _():
        @pl.when(sid < NS - 1)
        def _():
            pltpu.sync_copy(z_hbm, acc.at[pl.ds(ir0, 664)])

        @pl.when(sid == NS - 1)
        def _():
            pltpu.sync_copy(z_hbm.at[pl.ds(0, 40)], acc.at[pl.ds(ir0, 40)])

    plsc.subcore_barrier()

    bufs = (rows0, rows1, rows2)
    gsems = (sem0, sem1, sem2)

    pltpu.sync_copy(eidx_hbm.at[0, wid, 0], sring.at[0])
    pltpu.sync_copy(eidx_hbm.at[1, wid, 0], dring.at[0])

    @pl.loop(0, NGRP)
    def _grp(gi):
        rb = gi % 2
        nx = gi + 1

        @pl.when(nx < NGRP)
        def _():
            pltpu.async_copy(eidx_hbm.at[0, wid, nx], sring.at[1 - rb], rsem0)
            pltpu.async_copy(eidx_hbm.at[1, wid, nx], dring.at[1 - rb], rsem1)

        pltpu.async_copy(h_hbm.at[sring.at[rb, 0]], rows0, sem0)
        pltpu.async_copy(h_hbm.at[sring.at[rb, 1]], rows1, sem1)
        for j in range(GRP):
            b = j % 3
            if j + 2 < GRP:
                n = (j + 2) % 3
                pltpu.async_copy(h_hbm.at[sring.at[rb, j + 2]], bufs[n], gsems[n])
            pltpu.make_async_copy(h_hbm.at[sring.at[rb, j]], bufs[b],
                                  gsems[b]).wait()
            pltpu.sync_copy(bufs[b], acc.at[dring.at[rb, j]], add=True)

        @pl.when(nx < NGRP)
        def _():
            pltpu.make_async_copy(eidx_hbm.at[0, wid, nx], sring.at[1 - rb],
                                  rsem0).wait()
            pltpu.make_async_copy(eidx_hbm.at[1, wid, nx], dring.at[1 - rb],
                                  rsem1).wait()

    plsc.subcore_barrier()

    r0 = sid * 664

    @pl.when(sid < NS - 1)
    def _():
        pltpu.sync_copy(acc.at[pl.ds(r0, 664)],
                        out_hbm.at[pl.ds(cid * N + r0, 664)])

    @pl.when(sid == NS - 1)
    def _():
        pltpu.sync_copy(acc.at[pl.ds(r0, 40)],
                        out_hbm.at[pl.ds(cid * N + r0, 40)])


_B = 5000
_G = N // _B


def _tc1a_body(x_ref, w_ref, b_ref, h_ref):
    h = jnp.dot(x_ref[...], w_ref[...], preferred_element_type=jnp.float32)
    h_ref[...] = h + b_ref[...]


def _tc1a(x, W1, b1r):
    return pl.pallas_call(
        _tc1a_body,
        grid=(_G,),
        in_specs=[
            pl.BlockSpec((_B, D), lambda i: (i, 0)),
            pl.BlockSpec((D, D), lambda i: (0, 0)),
            pl.BlockSpec((1, D), lambda i: (0, 0)),
        ],
        out_specs=pl.BlockSpec((_B, D), lambda i: (i, 0)),
        out_shape=jax.ShapeDtypeStruct((N, D), jnp.float32),
    )(x, W1, b1r)


def _tc1b_body(xw_ref, d0_ref, d1_ref, h_ref, n_ref):
    norm = lax.rsqrt(d0_ref[...] + d1_ref[...] + 1.0)
    h_ref[...] = xw_ref[...] * norm
    n_ref[...] = jnp.broadcast_to(norm, (_B, D))


def _tc1b(xw, d0, d1):
    return pl.pallas_call(
        _tc1b_body,
        grid=(_G,),
        in_specs=[
            pl.BlockSpec((_B, D), lambda i: (i, 0)),
            pl.BlockSpec((_B, 1), lambda i: (i, 0)),
            pl.BlockSpec((_B, 1), lambda i: (i, 0)),
        ],
        out_specs=[
            pl.BlockSpec((_B, D), lambda i: (i, 0)),
            pl.BlockSpec((_B, D), lambda i: (i, 0)),
        ],
        out_shape=[
            jax.ShapeDtypeStruct((N, D), jnp.float32),
            jax.ShapeDtypeStruct((N, D), jnp.float32),
        ],
    )(xw, d0, d1)


def _tc2_body(s0_ref, s1_ref, n_ref, w_ref, b_ref, o_ref):
    norm = n_ref[...]
    h = jnp.maximum((s0_ref[...] + s1_ref[...]) * norm, 0.0)
    o = jnp.dot(h, w_ref[...], preferred_element_type=jnp.float32)
    o_ref[...] = (o + b_ref[...]) * norm


def _tc2(s, normc, W2, b2r):
    return pl.pallas_call(
        _tc2_body,
        grid=(_G,),
        in_specs=[
            pl.BlockSpec((_B, D), lambda i: (i, 0)),
            pl.BlockSpec((_B, D), lambda i: (_G + i, 0)),
            pl.BlockSpec((_B, D), lambda i: (i, 0)),
            pl.BlockSpec((D, D), lambda i: (0, 0)),
            pl.BlockSpec((1, D), lambda i: (0, 0)),
        ],
        out_specs=pl.BlockSpec((_B, D), lambda i: (i, 0)),
        out_shape=jax.ShapeDtypeStruct((N, D), jnp.float32),
    )(s, s, normc, W2, b2r)


def _tc3_body(s0_ref, s1_ref, n_ref, wz_ref, bz_ref, wl_ref, bl_ref,
              wt_ref, bt_ref, y_ref):
    x_ori = (s0_ref[...] + s1_ref[...]) * n_ref[...]
    z = jnp.dot(x_ori, wz_ref[...], preferred_element_type=jnp.float32)
    z = jnp.maximum(z + bz_ref[...], 0.0)
    t = jnp.dot(z, wl_ref[...], preferred_element_type=jnp.float32) + bl_ref[...]
    y_ref[...] = jnp.dot(t, wt_ref[...], preferred_element_type=jnp.float32) + bt_ref[...]


def _tc3(s, normc, Wz1, bz1r, Wl1, bl1r, Wtp, btp):
    return pl.pallas_call(
        _tc3_body,
        grid=(_G,),
        in_specs=[
            pl.BlockSpec((_B, D), lambda i: (i, 0)),
            pl.BlockSpec((_B, D), lambda i: (_G + i, 0)),
            pl.BlockSpec((_B, D), lambda i: (i, 0)),
            pl.BlockSpec((D, Z), lambda i: (0, 0)),
            pl.BlockSpec((1, Z), lambda i: (0, 0)),
            pl.BlockSpec((Z, H), lambda i: (0, 0)),
            pl.BlockSpec((1, H), lambda i: (0, 0)),
            pl.BlockSpec((H, C), lambda i: (0, 0)),
            pl.BlockSpec((1, C), lambda i: (0, 0)),
        ],
        out_specs=pl.BlockSpec((_B, C), lambda i: (i, 0)),
        out_shape=jax.ShapeDtypeStruct((N, C), jnp.float32),
    )(s, s, normc, Wz1, bz1r, Wl1, bl1r, Wtp, btp)


def kernel(x, edge_index, data, W1, b1, W2, b2, Wz1, bz1, Wl1, bl1, Wt, bt):
    eidxr = edge_index.reshape(2, NW, NGRP, GRP, K)

    degp = _sc_deg(eidxr)
    d0 = degp[:N].reshape(N, 1)
    d1 = degp[N:].reshape(N, 1)

    zrows = jnp.zeros((664, D), jnp.float32)
    xw = _tc1a(x, W1, b1.reshape(1, D))
    h1p, normc = _tc1b(xw, d0, d1)
    s1 = _sc_edge(h1p, eidxr, zrows)
    h2p = _tc2(s1, normc, W2, b2.reshape(1, D))
    s2 = _sc_edge(h2p, eidxr, zrows)

    return _tc3(s2, normc, Wz1, bz1.reshape(1, Z),
                Wl1, bl1.reshape(1, H), Wt, bt.reshape(1, C))


# speedup vs baseline: 1.0218x; 1.0218x over previous
import functools

import jax
import jax.numpy as jnp
from jax import lax
from jax.experimental import pallas as pl
from jax.experimental.pallas import tpu as pltpu
from jax.experimental.pallas import tpu_sc as plsc

N = 10000
D = 128
E = 320000
Z = 64
H = 64
C = 10

NC = 2
NS = 16
NW = NC * NS
EPW = E // NW
K = 100
NCH = EPW // K
GRP = 20
NGRP = NCH // GRP

_mesh = plsc.VectorSubcoreMesh(core_axis_name="c", subcore_axis_name="s")


@functools.partial(
    pl.kernel,
    out_type=jax.ShapeDtypeStruct((2 * N,), jnp.float32),
    mesh=_mesh,
    scratch_types=[
        pltpu.VMEM((NGRP, GRP, K), jnp.int32),
        pltpu.VMEM((128,), jnp.float32),
        pltpu.VMEM((1024,), jnp.float32),
        pltpu.VMEM_SHARED((N,), jnp.float32),
        pltpu.SemaphoreType.DMA,
    ],
)
def _sc_deg(eidx_hbm, out_hbm, didx, ones, zbuf, acc, dsem):
    cid = lax.axis_index("c")
    sid = lax.axis_index("s")
    wid = cid * NS + sid

    one16 = jnp.ones((16,), jnp.float32)
    zero16 = jnp.zeros((16,), jnp.float32)
    for j in range(8):
        ones[pl.ds(j * 16, 16)] = one16

    @pl.loop(0, 64)
    def _zero(i):
        zbuf[pl.ds(i * 16, 16)] = zero16

    @pl.when(sid < 10)
    def _():
        pltpu.sync_copy(zbuf.at[pl.ds(0, 1000)], acc.at[pl.ds(sid * 1000, 1000)])

    plsc.subcore_barrier()

    pltpu.sync_copy(eidx_hbm.at[1, wid], didx)

    @pl.loop(0, NGRP)
    def _edges(g):
        for j in range(GRP):
            pltpu.async_copy(ones.at[pl.ds(0, K)], acc.at[didx.at[g, j]],
                             dsem, add=True)
        for j in range(GRP):
            pltpu.make_async_copy(ones.at[pl.ds(0, K)], acc.at[didx.at[g, j]],
                                  dsem).wait()

    plsc.subcore_barrier()

    @pl.when(sid < 10)
    def _():
        pltpu.sync_copy(acc.at[pl.ds(sid * 1000, 1000)], zbuf.at[pl.ds(0, 1000)])
        pltpu.sync_copy(zbuf.at[pl.ds(0, 1000)],
                        out_hbm.at[pl.ds(cid * N + sid * 1000, 1000)])


@functools.partial(
    pl.kernel,
    out_type=jax.ShapeDtypeStruct((2 * N, D), jnp.float32),
    mesh=_mesh,
    scratch_types=[
        pltpu.VMEM((GRP, K), jnp.int32),
        pltpu.VMEM((GRP, K), jnp.int32),
        pltpu.VMEM((K, D), jnp.float32),
        pltpu.VMEM((K, D), jnp.float32),
        pltpu.VMEM((K, D), jnp.float32),
        pltpu.VMEM_SHARED((N, D), jnp.float32),
        pltpu.SemaphoreType.DMA,
        pltpu.SemaphoreType.DMA,
        pltpu.SemaphoreType.DMA,
    ],
)
def _sc_edge(h_hbm, eidx_hbm, z_hbm, out_hbm, sring, dring, rows0, rows1,
             rows2, acc, sem0, sem1, sem2):
    cid = lax.axis_index("c")
    sid = lax.axis_index("s")
    wid = cid * NS + sid

    ir0 = sid * 664

    @pl.when(cid == 0)
    def _():
        @pl.when(sid < NS - 1)
        def _():
            pltpu.sync_copy(h_hbm.at[pl.ds(ir0, 664)], acc.at[pl.ds(ir0, 664)])

        @pl.when(sid == NS - 1)
        def _():
            pltpu.sync_copy(h_hbm.at[pl.ds(ir0, 40)], acc.at[pl.ds(ir0, 40)])

    @pl.when(cid == 1)
    def _():
        @pl.when(sid < NS - 1)
        def _():
            pltpu.sync_copy(z_hbm, acc.at[pl.ds(ir0, 664)])

        @pl.when(sid == NS - 1)
        def _():
            pltpu.sync_copy(z_hbm.at[pl.ds(0, 40)], acc.at[pl.ds(ir0, 40)])

    plsc.subcore_barrier()

    bufs = (rows0, rows1, rows2)
    gsems = (sem0, sem1, sem2)

    @pl.loop(0, NGRP)
    def _grp(gi):
        pltpu.sync_copy(eidx_hbm.at[0, wid, gi], sring)
        pltpu.sync_copy(eidx_hbm.at[1, wid, gi], dring)
        pltpu.async_copy(h_hbm.at[sring.at[0]], rows0, sem0)
        pltpu.async_copy(h_hbm.at[sring.at[1]], rows1, sem1)
        for j in range(GRP):
            b = j % 3
            if j + 2 < GRP:
                n = (j + 2) % 3
                pltpu.async_copy(h_hbm.at[sring.at[j + 2]], bufs[n], gsems[n])
            pltpu.make_async_copy(h_hbm.at[sring.at[j]], bufs[b], gsems[b]).wait()
            pltpu.sync_copy(bufs[b], acc.at[dring.at[j]], add=True)

    plsc.subcore_barrier()

    r0 = sid * 664

    @pl.when(sid < NS - 1)
    def _():
        pltpu.sync_copy(acc.at[pl.ds(r0, 664)],
                        out_hbm.at[pl.ds(cid * N + r0, 664)])

    @pl.when(sid == NS - 1)
    def _():
        pltpu.sync_copy(acc.at[pl.ds(r0, 40)],
                        out_hbm.at[pl.ds(cid * N + r0, 40)])


_B = 5000
_G = N // _B


def _tc1a_body(x_ref, w_ref, b_ref, h_ref):
    h = jnp.dot(x_ref[...], w_ref[...], preferred_element_type=jnp.float32)
    h_ref[...] = h + b_ref[...]


def _tc1a(x, W1, b1r):
    return pl.pallas_call(
        _tc1a_body,
        grid=(_G,),
        in_specs=[
            pl.BlockSpec((_B, D), lambda i: (i, 0)),
            pl.BlockSpec((D, D), lambda i: (0, 0)),
            pl.BlockSpec((1, D), lambda i: (0, 0)),
        ],
        out_specs=pl.BlockSpec((_B, D), lambda i: (i, 0)),
        out_shape=jax.ShapeDtypeStruct((N, D), jnp.float32),
    )(x, W1, b1r)


def _tc1b_body(xw_ref, d0_ref, d1_ref, h_ref, n_ref):
    norm = lax.rsqrt(d0_ref[...] + d1_ref[...] + 1.0)
    h_ref[...] = xw_ref[...] * norm
    n_ref[...] = jnp.broadcast_to(norm, (_B, D))


def _tc1b(xw, d0, d1):
    return pl.pallas_call(
        _tc1b_body,
        grid=(_G,),
        in_specs=[
            pl.BlockSpec((_B, D), lambda i: (i, 0)),
            pl.BlockSpec((_B, 1), lambda i: (i, 0)),
            pl.BlockSpec((_B, 1), lambda i: (i, 0)),
        ],
        out_specs=[
            pl.BlockSpec((_B, D), lambda i: (i, 0)),
            pl.BlockSpec((_B, D), lambda i: (i, 0)),
        ],
        out_shape=[
            jax.ShapeDtypeStruct((N, D), jnp.float32),
            jax.ShapeDtypeStruct((N, D), jnp.float32),
        ],
    )(xw, d0, d1)


def _tc2_body(s0_ref, s1_ref, n_ref, w_ref, b_ref, o_ref):
    norm = n_ref[...]
    h = jnp.maximum((s0_ref[...] + s1_ref[...]) * norm, 0.0)
    o = jnp.dot(h, w_ref[...], preferred_element_type=jnp.float32)
    o_ref[...] = (o + b_ref[...]) * norm


def _tc2(s, normc, W2, b2r):
    return pl.pallas_call(
        _tc2_body,
        grid=(_G,),
        in_specs=[
            pl.BlockSpec((_B, D), lambda i: (i, 0)),
            pl.BlockSpec((_B, D), lambda i: (_G + i, 0)),
            pl.BlockSpec((_B, D), lambda i: (i, 0)),
            pl.BlockSpec((D, D), lambda i: (0, 0)),
            pl.BlockSpec((1, D), lambda i: (0, 0)),
        ],
        out_specs=pl.BlockSpec((_B, D), lambda i: (i, 0)),
        out_shape=jax.ShapeDtypeStruct((N, D), jnp.float32),
    )(s, s, normc, W2, b2r)


def _tc3_body(s0_ref, s1_ref, n_ref, wz_ref, bz_ref, wl_ref, bl_ref,
              wt_ref, bt_ref, y_ref):
    x_ori = (s0_ref[...] + s1_ref[...]) * n_ref[...]
    z = jnp.dot(x_ori, wz_ref[...], preferred_element_type=jnp.float32)
    z = jnp.maximum(z + bz_ref[...], 0.0)
    t = jnp.dot(z, wl_ref[...], preferred_element_type=jnp.float32) + bl_ref[...]
    y_ref[...] = jnp.dot(t, wt_ref[...], preferred_element_type=jnp.float32) + bt_ref[...]


def _tc3(s, normc, Wz1, bz1r, Wl1, bl1r, Wtp, btp):
    return pl.pallas_call(
        _tc3_body,
        grid=(_G,),
        in_specs=[
            pl.BlockSpec((_B, D), lambda i: (i, 0)),
            pl.BlockSpec((_B, D), lambda i: (_G + i, 0)),
            pl.BlockSpec((_B, D), lambda i: (i, 0)),
            pl.BlockSpec((D, Z), lambda i: (0, 0)),
            pl.BlockSpec((1, Z), lambda i: (0, 0)),
            pl.BlockSpec((Z, H), lambda i: (0, 0)),
            pl.BlockSpec((1, H), lambda i: (0, 0)),
            pl.BlockSpec((H, C), lambda i: (0, 0)),
            pl.BlockSpec((1, C), lambda i: (0, 0)),
        ],
        out_specs=pl.BlockSpec((_B, C), lambda i: (i, 0)),
        out_shape=jax.ShapeDtypeStruct((N, C), jnp.float32),
    )(s, s, normc, Wz1, bz1r, Wl1, bl1r, Wtp, btp)


def kernel(x, edge_index, data, W1, b1, W2, b2, Wz1, bz1, Wl1, bl1, Wt, bt):
    eidxr = edge_index.reshape(2, NW, NGRP, GRP, K)

    degp = _sc_deg(eidxr)
    d0 = degp[:N].reshape(N, 1)
    d1 = degp[N:].reshape(N, 1)

    zrows = jnp.zeros((664, D), jnp.float32)
    xw = _tc1a(x, W1, b1.reshape(1, D))
    h1p, normc = _tc1b(xw, d0, d1)
    s1 = _sc_edge(h1p, eidxr, zrows)
    h2p = _tc2(s1, normc, W2, b2.reshape(1, D))
    s2 = _sc_edge(h2p, eidxr, zrows)

    return _tc3(s2, normc, Wz1, bz1.reshape(1, Z),
                Wl1, bl1.reshape(1, H), Wt, bt.reshape(1, C))
